# Initial kernel scaffold; baseline (speedup 1.0000x reference)
import jax
import jax.numpy as jnp
from jax.experimental import pallas as pl


def kernel(x, edge_index_0, edge_index_1, edge_index_2, edge_index_3, edge_index_4, edge_index_5, batch, W1, as1, ad1, b1, W2, as2, ad2, b2, ln1_g, ln1_b, ln2_g, ln2_b, q_pool, Wp, bp):
    raise NotImplementedError("write your pallas kernel here")


# speedup vs baseline: 1.1248x; 1.1248x over previous
import functools
import jax
import jax.numpy as jnp
from jax.experimental import pallas as pl
from jax.experimental.pallas import tpu as pltpu

_N = 50000
_E = 100000
_D = 128
_HEADS = 4
_OC = 32
_R = 6
_NG = 64
_BLK = 512
_NPAD = ((_N + _BLK - 1) // _BLK) * _BLK
_NB = _NPAD // _BLK
_RD = _R * _D


def _dense1_body(x_ref, w_ref, asm_ref, adm_ref, hs_ref, as_ref, ad_ref):
    x = x_ref[...]
    hs_ref[...] = jnp.dot(x, w_ref[...], preferred_element_type=jnp.float32)
    as_ref[...] = jnp.dot(x, asm_ref[...], preferred_element_type=jnp.float32)
    ad_ref[...] = jnp.dot(x, adm_ref[...], preferred_element_type=jnp.float32)


def _dense1(x, wall, asm, adm):
    return pl.pallas_call(
        _dense1_body,
        grid=(_NB,),
        in_specs=[
            pl.BlockSpec((_BLK, _D), lambda i: (i, 0)),
            pl.BlockSpec((_D, _RD), lambda i: (0, 0)),
            pl.BlockSpec((_D, _D), lambda i: (0, 0)),
            pl.BlockSpec((_D, _D), lambda i: (0, 0)),
        ],
        out_specs=[
            pl.BlockSpec((_BLK, _RD), lambda i: (i, 0)),
            pl.BlockSpec((_BLK, _D), lambda i: (i, 0)),
            pl.BlockSpec((_BLK, _D), lambda i: (i, 0)),
        ],
        out_shape=[
            jax.ShapeDtypeStruct((_NPAD, _RD), jnp.float32),
            jax.ShapeDtypeStruct((_NPAD, _D), jnp.float32),
            jax.ShapeDtypeStruct((_NPAD, _D), jnp.float32),
        ],
    )(x, wall, asm, adm)


def _dense2_body(m_ref, bsum_ref, g_ref, b_ref, w_ref, asm_ref, adm_ref,
                 h_ref, hs_ref, as_ref, ad_ref):
    t = m_ref[...] + bsum_ref[...]
    t = t / (1.0 + jnp.abs(t))
    mu = jnp.mean(t, axis=-1, keepdims=True)
    var = jnp.mean((t - mu) ** 2, axis=-1, keepdims=True)
    ln = (t - mu) * jax.lax.rsqrt(var + 1e-5) * g_ref[...] + b_ref[...]
    h_ref[...] = ln
    hs_ref[...] = jnp.dot(ln, w_ref[...], preferred_element_type=jnp.float32)
    as_ref[...] = jnp.dot(ln, asm_ref[...], preferred_element_type=jnp.float32)
    ad_ref[...] = jnp.dot(ln, adm_ref[...], preferred_element_type=jnp.float32)


def _dense2(msum, bsum, g, b, wall, asm, adm):
    return pl.pallas_call(
        _dense2_body,
        grid=(_NB,),
        in_specs=[
            pl.BlockSpec((_BLK, _D), lambda i: (i, 0)),
            pl.BlockSpec((1, _D), lambda i: (0, 0)),
            pl.BlockSpec((1, _D), lambda i: (0, 0)),
            pl.BlockSpec((1, _D), lambda i: (0, 0)),
            pl.BlockSpec((_D, _RD), lambda i: (0, 0)),
            pl.BlockSpec((_D, _D), lambda i: (0, 0)),
            pl.BlockSpec((_D, _D), lambda i: (0, 0)),
        ],
        out_specs=[
            pl.BlockSpec((_BLK, _D), lambda i: (i, 0)),
            pl.BlockSpec((_BLK, _RD), lambda i: (i, 0)),
            pl.BlockSpec((_BLK, _D), lambda i: (i, 0)),
            pl.BlockSpec((_BLK, _D), lambda i: (i, 0)),
        ],
        out_shape=[
            jax.ShapeDtypeStruct((_NPAD, _D), jnp.float32),
            jax.ShapeDtypeStruct((_NPAD, _RD), jnp.float32),
            jax.ShapeDtypeStruct((_NPAD, _D), jnp.float32),
            jax.ShapeDtypeStruct((_NPAD, _D), jnp.float32),
        ],
    )(msum, bsum, g, b, wall, asm, adm)


def _pool_body(m_ref, bsum_ref, g_ref, b_ref, q_ref, batch_ref, wp_ref,
               bp_ref, out_ref, mmax, num_acc, den_acc):
    p = pl.program_id(0)
    j = pl.program_id(1)

    t = m_ref[...] + bsum_ref[...]
    t = t / (1.0 + jnp.abs(t))
    mu = jnp.mean(t, axis=-1, keepdims=True)
    var = jnp.mean((t - mu) ** 2, axis=-1, keepdims=True)
    h2 = (t - mu) * jax.lax.rsqrt(var + 1e-5) * g_ref[...] + b_ref[...]
    s = jnp.sum(h2 * q_ref[...], axis=-1, keepdims=True)

    gids = jax.lax.broadcasted_iota(jnp.int32, (_BLK, _NG), 1)
    oh = (batch_ref[...] == gids).astype(jnp.float32)

    @pl.when((p == 0) & (j == 0))
    def _():
        mmax[...] = jnp.full((1, _NG), -1e30, jnp.float32)

    @pl.when(p == 0)
    def _():
        part = jnp.max(jnp.where(oh > 0, s, -1e30), axis=0, keepdims=True)
        mmax[...] = jnp.maximum(mmax[...], part)

    @pl.when((p == 1) & (j == 0))
    def _():
        num_acc[...] = jnp.zeros((_NG, _D), jnp.float32)
        den_acc[...] = jnp.zeros((_NG, _D), jnp.float32)

    @pl.when(p == 1)
    def _():
        m_row = jnp.sum(oh * mmax[...], axis=1, keepdims=True)
        w = jnp.exp(s - m_row) * (oh.sum(axis=1, keepdims=True))
        num_acc[...] += jax.lax.dot_general(
            oh, w * h2, (((0,), (0,)), ((), ())),
            preferred_element_type=jnp.float32)
        den_acc[...] += jax.lax.dot_general(
            oh, jnp.broadcast_to(w, (_BLK, _D)), (((0,), (0,)), ((), ())),
            preferred_element_type=jnp.float32)

    @pl.when((p == 1) & (j == _NB - 1))
    def _():
        pooled = num_acc[...] / (den_acc[...] + 1e-16)
        out_ref[...] = jnp.dot(pooled, wp_ref[...],
                               preferred_element_type=jnp.float32) + bp_ref[...]


def _pool(msum2, bsum, g, b, q, batch2d, wp, bp):
    return pl.pallas_call(
        _pool_body,
        grid=(2, _NB),
        in_specs=[
            pl.BlockSpec((_BLK, _D), lambda p, j: (j, 0)),
            pl.BlockSpec((1, _D), lambda p, j: (0, 0)),
            pl.BlockSpec((1, _D), lambda p, j: (0, 0)),
            pl.BlockSpec((1, _D), lambda p, j: (0, 0)),
            pl.BlockSpec((1, _D), lambda p, j: (0, 0)),
            pl.BlockSpec((_BLK, 1), lambda p, j: (j, 0)),
            pl.BlockSpec((_D, _D), lambda p, j: (0, 0)),
            pl.BlockSpec((1, _D), lambda p, j: (0, 0)),
        ],
        out_specs=pl.BlockSpec((_NG, _D), lambda p, j: (0, 0)),
        out_shape=jax.ShapeDtypeStruct((_NG, _D), jnp.float32),
        scratch_shapes=[
            pltpu.VMEM((1, _NG), jnp.float32),
            pltpu.VMEM((_NG, _D), jnp.float32),
            pltpu.VMEM((_NG, _D), jnp.float32),
        ],
    )(msum2, bsum, g, b, q, batch2d, wp, bp)


def _edge_phase(hs, asrc, adst, eis):
    msum = jnp.zeros((_N, _D), jnp.float32)
    for r in range(_R):
        src, dst = eis[r][0], eis[r][1]
        e = asrc[src, r * _HEADS:(r + 1) * _HEADS] + \
            adst[dst, r * _HEADS:(r + 1) * _HEADS]
        e = jnp.where(e > 0, e, 0.2 * e)
        ex = jnp.exp(e)
        den = jax.ops.segment_sum(ex, dst, num_segments=_N)
        alpha = ex / (den[dst] + 1e-16)
        hblk = hs[:, r * _D:(r + 1) * _D]
        msg = hblk[src].reshape(_E, _HEADS, _OC) * alpha[:, :, None]
        out = jax.ops.segment_sum(msg, dst, num_segments=_N)
        msum = msum + out.reshape(_N, _D)
    return msum


def _fold_alpha(W, a):
    Wr = W.reshape(_R, _D, _HEADS, _OC)
    A = jnp.einsum('rdho,rho->drh', Wr, a).reshape(_D, _R * _HEADS)
    return jnp.pad(A, ((0, 0), (0, _D - _R * _HEADS)))


def kernel(x, edge_index_0, edge_index_1, edge_index_2, edge_index_3,
           edge_index_4, edge_index_5, batch, W1, as1, ad1, b1, W2, as2,
           ad2, b2, ln1_g, ln1_b, ln2_g, ln2_b, q_pool, Wp, bp):
    eis = (edge_index_0, edge_index_1, edge_index_2, edge_index_3,
           edge_index_4, edge_index_5)

    w1all = W1.transpose(1, 0, 2).reshape(_D, _RD)
    w2all = W2.transpose(1, 0, 2).reshape(_D, _RD)
    a1s, a1d = _fold_alpha(W1, as1), _fold_alpha(W1, ad1)
    a2s, a2d = _fold_alpha(W2, as2), _fold_alpha(W2, ad2)
    b1sum = jnp.sum(b1, axis=0)[None, :]
    b2sum = jnp.sum(b2, axis=0)[None, :]

    xp = jnp.pad(x, ((0, _NPAD - _N), (0, 0)))
    batch_pad = jnp.pad(batch.astype(jnp.int32), (0, _NPAD - _N),
                        constant_values=_NG)[:, None]

    hs1, as1p, ad1p = _dense1(xp, w1all, a1s, a1d)
    msum1 = _edge_phase(hs1[:_N], as1p[:_N], ad1p[:_N], eis)
    msum1 = jnp.pad(msum1, ((0, _NPAD - _N), (0, 0)))

    h1, hs2, as2p, ad2p = _dense2(msum1, b1sum, ln1_g[None, :],
                                  ln1_b[None, :], w2all, a2s, a2d)
    del h1
    msum2 = _edge_phase(hs2[:_N], as2p[:_N], ad2p[:_N], eis)
    msum2 = jnp.pad(msum2, ((0, _NPAD - _N), (0, 0)))

    return _pool(msum2, b2sum, ln2_g[None, :], ln2_b[None, :],
                 q_pool[None, :], batch_pad, Wp, bp[None, :])
